# baseline (speedup 1.0000x reference)
import functools

import jax
import jax.numpy as jnp
import numpy as np
from jax import lax
from jax.experimental import pallas as pl
from jax.experimental.pallas import tpu as pltpu
from jax.experimental.pallas import tpu_sc as plsc

VOCAB = 100000
D = 768
B = 4
S = 2048
N = B * S

NC, NS, L = 2, 16, 16
NW = NC * NS
PER_W = N // NW
R = 64
NCHUNK = PER_W // R
DV = D // L


def _pe_table(max_len, d_model):
    pos = np.arange(max_len, dtype=np.float32)[:, None]
    i = np.arange(d_model, dtype=np.float32)[None, :]
    angle_rates = 1.0 / np.power(10000.0, (2.0 * np.floor(i / 2.0)) / float(d_model))
    angles = pos * angle_rates
    pe = np.zeros((max_len, d_model), dtype=np.float32)
    pe[:, 0::2] = np.sin(angles[:, 0::2])
    pe[:, 1::2] = np.cos(angles[:, 1::2])
    return pe


_PE = _pe_table(S, D)

_mesh = plsc.VectorSubcoreMesh(core_axis_name="c", subcore_axis_name="s")


@functools.partial(
    pl.kernel,
    out_type=jax.ShapeDtypeStruct((N, D), jnp.float32),
    mesh=_mesh,
    scratch_types=[
        pltpu.VMEM((R,), jnp.int32),
        pltpu.VMEM((R, D), jnp.float32),
        pltpu.VMEM((R, D), jnp.float32),
        pltpu.SemaphoreType.DMA,
    ],
)
def _embed_kernel(table_hbm, idx_hbm, pe_hbm, out_hbm, idx_v, tok_v, pe_v, sem):
    wid = lax.axis_index("s") * NC + lax.axis_index("c")
    base = wid * PER_W
    s_base = (wid % (S // PER_W)) * PER_W

    for c in range(NCHUNK):
        pltpu.sync_copy(idx_hbm.at[pl.ds(base + c * R, R)], idx_v)
        gather = pltpu.async_copy(table_hbm.at[idx_v], tok_v, sem)
        pltpu.sync_copy(pe_hbm.at[pl.ds(s_base + c * R, R)], pe_v)
        gather.wait()

        def add_row(r, carry):
            for k in range(DV):
                sl = pl.ds(k * L, L)
                tok_v[r, sl] = tok_v[r, sl] + pe_v[r, sl]
            return carry

        lax.fori_loop(0, R, add_row, 0)
        pltpu.sync_copy(tok_v, out_hbm.at[pl.ds(base + c * R, R)])


@jax.jit
def _embed(x, table):
    idx = x.reshape(N)
    out = _embed_kernel(table, idx, jnp.asarray(_PE))
    return out.reshape(B, S, D)


def kernel(x, table):
    return _embed(x, table)


# speedup vs baseline: 1.0388x; 1.0388x over previous
import functools

import jax
import jax.numpy as jnp
import numpy as np
from jax import lax
from jax.experimental import pallas as pl
from jax.experimental.pallas import tpu as pltpu
from jax.experimental.pallas import tpu_sc as plsc

VOCAB = 100000
D = 768
B = 4
S = 2048
N = B * S

NC, NS, L = 2, 16, 16
NW = NC * NS
S_W = S // NW
R = 32
NCHUNK = (B * S_W) // R
DV = D // L
NBUF = 3


def _pe_table(max_len, d_model):
    pos = np.arange(max_len, dtype=np.float32)[:, None]
    i = np.arange(d_model, dtype=np.float32)[None, :]
    angle_rates = 1.0 / np.power(10000.0, (2.0 * np.floor(i / 2.0)) / float(d_model))
    angles = pos * angle_rates
    pe = np.zeros((max_len, d_model), dtype=np.float32)
    pe[:, 0::2] = np.sin(angles[:, 0::2])
    pe[:, 1::2] = np.cos(angles[:, 1::2])
    return pe


_PE = _pe_table(S, D)

_mesh = plsc.VectorSubcoreMesh(core_axis_name="c", subcore_axis_name="s")


@functools.partial(
    pl.kernel,
    out_type=jax.ShapeDtypeStruct((N, D), jnp.float32),
    mesh=_mesh,
    scratch_types=[
        pltpu.VMEM((S_W, D), jnp.float32),
        [pltpu.VMEM((R,), jnp.int32) for _ in range(NBUF)],
        [pltpu.VMEM((R, D), jnp.float32) for _ in range(NBUF)],
        [pltpu.SemaphoreType.DMA for _ in range(NBUF)],
        [pltpu.SemaphoreType.DMA for _ in range(NBUF)],
    ],
)
def _embed_kernel(table_hbm, idx_hbm, pe_hbm, out_hbm,
                  pe_v, idx_v, tok_v, sem_in, sem_out):
    wid = lax.axis_index("s") * NC + lax.axis_index("c")
    s0 = wid * S_W

    def chunk_base(i):
        return (i // 2) * S + s0 + (i % 2) * R

    pltpu.sync_copy(pe_hbm.at[pl.ds(s0, S_W)], pe_v)

    def issue_gather(i):
        j = i % NBUF
        pltpu.sync_copy(idx_hbm.at[pl.ds(chunk_base(i), R)], idx_v[j])
        return pltpu.async_copy(table_hbm.at[idx_v[j]], tok_v[j], sem_in[j])

    gathers = [issue_gather(0), issue_gather(1)]
    scatters = [None] * NCHUNK

    for i in range(NCHUNK):
        j = i % NBUF
        gathers[i].wait()
        pe_off = (i % 2) * R

        def add_row(r, carry):
            for k in range(DV):
                sl = pl.ds(k * L, L)
                tok_v[j][r, sl] = tok_v[j][r, sl] + pe_v[pe_off + r, sl]
            return carry

        lax.fori_loop(0, R, add_row, 0)
        scatters[i] = pltpu.async_copy(
            tok_v[j], out_hbm.at[pl.ds(chunk_base(i), R)], sem_out[j])
        if i + 2 < NCHUNK:
            if i >= 1:
                scatters[i - 1].wait()
            gathers.append(issue_gather(i + 2))

    scatters[NCHUNK - 2].wait()
    scatters[NCHUNK - 1].wait()


@jax.jit
def _embed(x, table):
    idx = x.reshape(N)
    out = _embed_kernel(table, idx, jnp.asarray(_PE))
    return out.reshape(B, S, D)


def kernel(x, table):
    return _embed(x, table)


# speedup vs baseline: 1.2882x; 1.2401x over previous
import functools

import jax
import jax.numpy as jnp
import numpy as np
from jax import lax
from jax.experimental import pallas as pl
from jax.experimental.pallas import tpu as pltpu
from jax.experimental.pallas import tpu_sc as plsc

VOCAB = 100000
D = 768
B = 4
S = 2048
N = B * S

NC, NS, L = 2, 16, 16
NW = NC * NS
S_W = S // NW
R = 32
NCHUNK = (B * S_W) // R
DV = D // L
NBUF = 3


def _pe_table(max_len, d_model):
    pos = np.arange(max_len, dtype=np.float32)[:, None]
    i = np.arange(d_model, dtype=np.float32)[None, :]
    angle_rates = 1.0 / np.power(10000.0, (2.0 * np.floor(i / 2.0)) / float(d_model))
    angles = pos * angle_rates
    pe = np.zeros((max_len, d_model), dtype=np.float32)
    pe[:, 0::2] = np.sin(angles[:, 0::2])
    pe[:, 1::2] = np.cos(angles[:, 1::2])
    return pe


_PE = _pe_table(S, D)

_mesh = plsc.VectorSubcoreMesh(core_axis_name="c", subcore_axis_name="s")


@functools.partial(
    pl.kernel,
    out_type=jax.ShapeDtypeStruct((N, D), jnp.float32),
    mesh=_mesh,
    scratch_types=[
        pltpu.VMEM((S_W, D), jnp.float32),
        pltpu.VMEM((B, S_W), jnp.int32),
        [pltpu.VMEM((R, D), jnp.float32) for _ in range(NBUF)],
        pltpu.SemaphoreType.DMA,
        [pltpu.SemaphoreType.DMA for _ in range(NBUF)],
        [pltpu.SemaphoreType.DMA for _ in range(NBUF)],
    ],
)
def _embed_kernel(table_hbm, idx_hbm, pe_hbm, out_hbm,
                  pe_v, idx_v, tok_v, sem_pre, sem_in, sem_out):
    wid = lax.axis_index("s") * NC + lax.axis_index("c")
    s0 = wid * S_W

    idx_copies = [
        pltpu.async_copy(idx_hbm.at[pl.ds(b * S + s0, S_W)], idx_v.at[b], sem_pre)
        for b in range(B)
    ]
    pe_copy = pltpu.async_copy(pe_hbm.at[pl.ds(s0, S_W)], pe_v, sem_pre)
    for c in idx_copies:
        c.wait()

    def issue_gather(i):
        b, h = i // 2, i % 2
        return pltpu.async_copy(
            table_hbm.at[idx_v.at[b, pl.ds(h * R, R)]], tok_v[i % NBUF],
            sem_in[i % NBUF])

    gathers = [issue_gather(0), issue_gather(1)]
    scatters = [None] * NCHUNK
    pe_copy.wait()

    for i in range(NCHUNK):
        j = i % NBUF
        gathers[i].wait()
        pe_off = (i % 2) * R
        tok = tok_v[j]

        @plsc.parallel_loop(0, R, unroll=2)
        def add_row(r):
            for k in range(DV):
                sl = pl.ds(k * L, L)
                tok[r, sl] = tok[r, sl] + pe_v[pe_off + r, sl]

        b, h = i // 2, i % 2
        scatters[i] = pltpu.async_copy(
            tok_v[j], out_hbm.at[pl.ds(b * S + s0 + h * R, R)], sem_out[j])
        if i + 2 < NCHUNK:
            if i >= 1:
                scatters[i - 1].wait()
            gathers.append(issue_gather(i + 2))

    scatters[NCHUNK - 2].wait()
    scatters[NCHUNK - 1].wait()


@jax.jit
def _embed(x, table):
    idx = x.reshape(N)
    out = _embed_kernel(table, idx, jnp.asarray(_PE))
    return out.reshape(B, S, D)


def kernel(x, table):
    return _embed(x, table)


# speedup vs baseline: 1.3765x; 1.0685x over previous
import functools

import jax
import jax.numpy as jnp
import numpy as np
from jax import lax
from jax.experimental import pallas as pl
from jax.experimental.pallas import tpu as pltpu
from jax.experimental.pallas import tpu_sc as plsc

VOCAB = 100000
D = 768
B = 4
S = 2048
N = B * S

NC, NS, L = 2, 16, 16
NW = NC * NS
S_W = S // NW
R = 32
NCHUNK = (B * S_W) // R
DV2 = D // (2 * L)
NBUF = 4


def _pe_table(max_len, d_model):
    pos = np.arange(max_len, dtype=np.float32)[:, None]
    i = np.arange(d_model, dtype=np.float32)[None, :]
    angle_rates = 1.0 / np.power(10000.0, (2.0 * np.floor(i / 2.0)) / float(d_model))
    angles = pos * angle_rates
    pe = np.zeros((max_len, d_model), dtype=np.float32)
    pe[:, 0::2] = np.sin(angles[:, 0::2])
    pe[:, 1::2] = np.cos(angles[:, 1::2])
    return pe


def _pe_packed(max_len, d_model):
    pe = _pe_table(max_len, d_model).reshape(max_len, d_model // 32, 2, 16)
    out = np.empty((max_len, d_model // 32, 32), dtype=np.float32)
    out[:, :, 0::2] = pe[:, :, 0, :]
    out[:, :, 1::2] = pe[:, :, 1, :]
    return out.reshape(max_len, d_model)


_PE_PACKED = _pe_packed(S, D)

_mesh = plsc.VectorSubcoreMesh(core_axis_name="c", subcore_axis_name="s")


@functools.partial(
    pl.kernel,
    out_type=jax.ShapeDtypeStruct((N, D), jnp.float32),
    mesh=_mesh,
    compiler_params=pltpu.CompilerParams(needs_layout_passes=False),
    scratch_types=[
        pltpu.VMEM((S_W, D // 2), jnp.int32),
        pltpu.VMEM((B, S_W), jnp.int32),
        [pltpu.VMEM((R, D), jnp.float32) for _ in range(NBUF)],
        pltpu.SemaphoreType.DMA,
        [pltpu.SemaphoreType.DMA for _ in range(NBUF)],
        [pltpu.SemaphoreType.DMA for _ in range(NBUF)],
    ],
)
def _embed_kernel(table_hbm, idx_hbm, pe_hbm, out_hbm,
                  pe_v, idx_v, tok_v, sem_pre, sem_in, sem_out):
    wid = lax.axis_index("s") * NC + lax.axis_index("c")
    s0 = wid * S_W

    idx_copies = [
        pltpu.async_copy(idx_hbm.at[b, pl.ds(s0, S_W)], idx_v.at[b], sem_pre)
        for b in range(B)
    ]
    pe_copy = pltpu.async_copy(pe_hbm.at[pl.ds(s0, S_W)], pe_v, sem_pre)
    for c in idx_copies:
        c.wait()

    def issue_gather(i):
        b, h = i // 2, i % 2
        return pltpu.async_copy(
            table_hbm.at[idx_v.at[b, pl.ds(h * R, R)]], tok_v[i % NBUF],
            sem_in[i % NBUF])

    gathers = [issue_gather(0), issue_gather(1)]
    scatters = [None] * NCHUNK
    pe_copy.wait()

    for i in range(NCHUNK):
        j = i % NBUF
        gathers[i].wait()
        pe_off = (i % 2) * R
        tok = tok_v[j]

        @plsc.parallel_loop(0, R)
        def add_row(r):
            prow = pe_off + r
            for k in range(DV2):
                v = pe_v[prow, pl.ds(k * L, L)]
                lo = plsc.bitcast(v << 16, jnp.float32)
                hi = plsc.bitcast(v & jnp.int32(-65536), jnp.float32)
                sl_lo = pl.ds(k * 2 * L, L)
                sl_hi = pl.ds(k * 2 * L + L, L)
                tok[r, sl_lo] = tok[r, sl_lo] + lo
                tok[r, sl_hi] = tok[r, sl_hi] + hi

        b, h = i // 2, i % 2
        scatters[i] = pltpu.async_copy(
            tok_v[j], out_hbm.at[pl.ds(b * S + s0 + h * R, R)], sem_out[j])
        if i + 2 < NCHUNK:
            if i >= 2:
                scatters[i - 2].wait()
            gathers.append(issue_gather(i + 2))

    for i in range(NCHUNK - 4, NCHUNK):
        scatters[i].wait()


@jax.jit
def _embed(x, table):
    pe_bf = jnp.asarray(_PE_PACKED).astype(jnp.bfloat16).reshape(S, D // 2, 2)
    pe = lax.bitcast_convert_type(pe_bf, jnp.int32)
    out = _embed_kernel(table, x, pe)
    return out.reshape(B, S, D)


def kernel(x, table):
    return _embed(x, table)


# speedup vs baseline: 1.3776x; 1.0008x over previous
import functools

import jax
import jax.numpy as jnp
import numpy as np
from jax import lax
from jax.experimental import pallas as pl
from jax.experimental.pallas import tpu as pltpu
from jax.experimental.pallas import tpu_sc as plsc

VOCAB = 100000
D = 768
B = 4
S = 2048
N = B * S

NC, NS, L = 2, 16, 16
NW = NC * NS
S_W = S // NW
R = 32
NCHUNK = (B * S_W) // R
DV2 = D // (2 * L)
NBUF = 4


def _pe_table(max_len, d_model):
    pos = np.arange(max_len, dtype=np.float32)[:, None]
    i = np.arange(d_model, dtype=np.float32)[None, :]
    angle_rates = 1.0 / np.power(10000.0, (2.0 * np.floor(i / 2.0)) / float(d_model))
    angles = pos * angle_rates
    pe = np.zeros((max_len, d_model), dtype=np.float32)
    pe[:, 0::2] = np.sin(angles[:, 0::2])
    pe[:, 1::2] = np.cos(angles[:, 1::2])
    return pe


def _pe_packed(max_len, d_model):
    pe = _pe_table(max_len, d_model).reshape(max_len, d_model // 32, 2, 16)
    out = np.empty((max_len, d_model // 32, 32), dtype=np.float32)
    out[:, :, 0::2] = pe[:, :, 0, :]
    out[:, :, 1::2] = pe[:, :, 1, :]
    return out.reshape(max_len, d_model)


def _pe_packed_i32(max_len, d_model):
    import ml_dtypes
    pe_bf = _pe_packed(max_len, d_model).astype(ml_dtypes.bfloat16)
    return pe_bf.view(np.uint32).astype(np.int32).reshape(max_len, d_model // 2)


_PE_I32 = _pe_packed_i32(S, D)

_mesh = plsc.VectorSubcoreMesh(core_axis_name="c", subcore_axis_name="s")


@functools.partial(
    pl.kernel,
    out_type=jax.ShapeDtypeStruct((N, D), jnp.float32),
    mesh=_mesh,
    compiler_params=pltpu.CompilerParams(needs_layout_passes=False),
    scratch_types=[
        pltpu.VMEM((S_W, D // 2), jnp.int32),
        pltpu.VMEM((B, S_W), jnp.int32),
        [pltpu.VMEM((R, D), jnp.float32) for _ in range(NBUF)],
        pltpu.SemaphoreType.DMA,
        [pltpu.SemaphoreType.DMA for _ in range(NBUF)],
        [pltpu.SemaphoreType.DMA for _ in range(NBUF)],
    ],
)
def _embed_kernel(table_hbm, idx_hbm, pe_hbm, out_hbm,
                  pe_v, idx_v, tok_v, sem_pre, sem_in, sem_out):
    wid = lax.axis_index("s") * NC + lax.axis_index("c")
    s0 = wid * S_W

    idx_copies = [
        pltpu.async_copy(idx_hbm.at[b, pl.ds(s0, S_W)], idx_v.at[b], sem_pre)
        for b in range(B)
    ]
    pe_copy = pltpu.async_copy(pe_hbm.at[pl.ds(s0, S_W)], pe_v, sem_pre)
    for c in idx_copies:
        c.wait()

    def issue_gather(i):
        b, h = i // 2, i % 2
        return pltpu.async_copy(
            table_hbm.at[idx_v.at[b, pl.ds(h * R, R)]], tok_v[i % NBUF],
            sem_in[i % NBUF])

    gathers = [issue_gather(0), issue_gather(1)]
    scatters = [None] * NCHUNK
    pe_copy.wait()

    for i in range(NCHUNK):
        j = i % NBUF
        gathers[i].wait()
        pe_off = (i % 2) * R
        tok = tok_v[j]

        @plsc.parallel_loop(0, R)
        def add_row(r):
            prow = pe_off + r
            for k in range(DV2):
                v = pe_v[prow, pl.ds(k * L, L)]
                lo = plsc.bitcast(v << 16, jnp.float32)
                hi = plsc.bitcast(v & jnp.int32(-65536), jnp.float32)
                sl_lo = pl.ds(k * 2 * L, L)
                sl_hi = pl.ds(k * 2 * L + L, L)
                tok[r, sl_lo] = tok[r, sl_lo] + lo
                tok[r, sl_hi] = tok[r, sl_hi] + hi

        b, h = i // 2, i % 2
        scatters[i] = pltpu.async_copy(
            tok_v[j], out_hbm.at[pl.ds(b * S + s0 + h * R, R)], sem_out[j])
        if i + 2 < NCHUNK:
            if i >= 2:
                scatters[i - 2].wait()
            gathers.append(issue_gather(i + 2))

    for i in range(NCHUNK - 4, NCHUNK):
        scatters[i].wait()


@jax.jit
def _embed(x, table):
    pe = jnp.asarray(_PE_I32)
    out = _embed_kernel(table, x, pe)
    return out.reshape(B, S, D)


def kernel(x, table):
    return _embed(x, table)
